# baseline (speedup 1.0000x reference)
import functools
import math

import jax
import jax.numpy as jnp
from jax import lax
from jax.experimental import pallas as pl
from jax.experimental.pallas import tpu as pltpu
from jax.experimental.pallas import tpu_sc as plsc

B = 2
T = 4096
E = 2048
D = 128
NT = 8
TB = T // NT
NSEL = int(0.1 * T)
S = 512
_F32_INF_BITS = 0x7F800000
_HI = jax.lax.Precision.HIGHEST


def _body_a(idx_ref, wq_ref, qg_ref, pos_ref, idxf_ref, q_s, nrm_s, pos_s):
    t = pl.program_id(1)
    x = idx_ref[0]
    q = jnp.dot(x, wq_ref[...], preferred_element_type=jnp.float32)
    q_s[pl.ds(t * TB, TB), :] = q
    ones_row = jnp.ones((1, D), jnp.float32)
    n2 = jax.lax.dot_general(ones_row, q * q, (((1,), (1,)), ((), ())),
                             preferred_element_type=jnp.float32, precision=_HI)
    nrm_s[pl.ds(t, 1), :] = jnp.sqrt(n2)

    @pl.when(t == NT - 1)
    def _finish():
        norms = nrm_s[...]
        bits = jax.lax.bitcast_convert_type(norms, jnp.int32)

        def bs_body(_, carry):
            lo, hi = carry
            mid = lo + (hi - lo) // 2
            cnt = jnp.sum((bits > mid).astype(jnp.int32))
            big = cnt >= NSEL
            return jnp.where(big, mid, lo), jnp.where(big, hi, mid)

        lo, hi = jax.lax.fori_loop(
            0, 31, bs_body, (jnp.int32(-1), jnp.int32(_F32_INF_BITS)))
        thr = hi
        m_gt = bits > thr
        m_eq = bits == thr

        io_i = jax.lax.broadcasted_iota(jnp.int32, (TB, TB), 0)
        io_j = jax.lax.broadcasted_iota(jnp.int32, (TB, TB), 1)
        tri_l = (io_i <= io_j).astype(jnp.float32)
        ro_i = jax.lax.broadcasted_iota(jnp.int32, (NT, NT), 0)
        ro_j = jax.lax.broadcasted_iota(jnp.int32, (NT, NT), 1)
        tri_s = (ro_j < ro_i).astype(jnp.float32)

        def csum(mb):
            mf = mb.astype(jnp.float32)
            within = jnp.dot(mf, tri_l,
                             preferred_element_type=jnp.float32, precision=_HI)
            off = jnp.dot(tri_s, within[:, TB - 1:TB],
                          preferred_element_type=jnp.float32, precision=_HI)
            return within + off

        n_gt = jnp.sum(m_gt.astype(jnp.int32))
        need = (NSEL - n_gt).astype(jnp.float32)
        sel = m_gt | (m_eq & (csum(m_eq) <= need))
        pos = csum(sel) - 1.0
        pos_s[...] = jnp.where(sel, pos.astype(jnp.int32), -1)
        pos_ref[0] = pos_s[...]

        iota_s = jax.lax.broadcasted_iota(jnp.int32, (S, TB), 0)

        def build_gather(r, acc):
            qa, ia = acc
            pr = pos_s[pl.ds(r, 1), :]
            chunk = (pr == iota_s).astype(jnp.float32)
            rows = pl.ds(r * TB, TB)
            qa = qa + jnp.dot(chunk, q_s[rows, :],
                              preferred_element_type=jnp.float32)
            tcol = (jax.lax.broadcasted_iota(jnp.int32, (TB, 1), 0)
                    + r * TB).astype(jnp.float32)
            ia = ia + jnp.dot(chunk, tcol,
                              preferred_element_type=jnp.float32, precision=_HI)
            return qa, ia

        qg, idxcol = jax.lax.fori_loop(
            0, NT, build_gather,
            (jnp.zeros((S, D), jnp.float32), jnp.zeros((S, 1), jnp.float32)))
        qg_ref[0] = qg
        eye_s = (jax.lax.broadcasted_iota(jnp.int32, (S, S), 0) ==
                 jax.lax.broadcasted_iota(jnp.int32, (S, S), 1)
                 ).astype(jnp.float32)
        idxf_ref[0] = jax.lax.dot_general(
            idxcol, eye_s, (((0,), (0,)), ((), ())),
            preferred_element_type=jnp.float32, precision=_HI)


def _stage_a(index, Wq, interpret):
    return pl.pallas_call(
        _body_a,
        grid=(B, NT),
        in_specs=[
            pl.BlockSpec((1, TB, E), lambda b, t: (b, t, 0)),
            pl.BlockSpec((E, D), lambda b, t: (0, 0)),
        ],
        out_specs=[
            pl.BlockSpec((1, S, D), lambda b, t: (b, 0, 0)),
            pl.BlockSpec((1, NT, TB), lambda b, t: (b, 0, 0)),
            pl.BlockSpec((1, 1, S), lambda b, t: (b, 0, 0)),
        ],
        out_shape=[
            jax.ShapeDtypeStruct((B, S, D), jnp.float32),
            jax.ShapeDtypeStruct((B, NT, TB), jnp.int32),
            jax.ShapeDtypeStruct((B, 1, S), jnp.float32),
        ],
        scratch_shapes=[
            pltpu.VMEM((T, D), jnp.float32),
            pltpu.VMEM((NT, TB), jnp.float32),
            pltpu.VMEM((NT, TB), jnp.int32),
        ],
        interpret=interpret,
    )(index, Wq)


def _sc_gather(table, idx_flat):
    info = plsc.get_sparse_core_info()
    nc, ns = info.num_cores, info.num_subcores
    nw = nc * ns
    rows_per_w = (B * S) // nw
    mesh = plsc.VectorSubcoreMesh(core_axis_name="c", subcore_axis_name="s")

    @functools.partial(
        pl.kernel, mesh=mesh,
        out_type=jax.ShapeDtypeStruct((B * S, E), jnp.float32),
        scratch_types=[
            pltpu.VMEM((rows_per_w,), jnp.int32),
            pltpu.VMEM((rows_per_w, E), jnp.float32),
            pltpu.SemaphoreType.DMA,
        ],
    )
    def gk(table_hbm, idx_hbm, out_hbm, idx_v, rows_v, sem):
        wid = lax.axis_index("s") * nc + lax.axis_index("c")
        base = wid * rows_per_w
        pltpu.sync_copy(idx_hbm.at[pl.ds(base, rows_per_w)], idx_v)
        pltpu.async_copy(table_hbm.at[idx_v], rows_v, sem).wait()
        pltpu.sync_copy(rows_v, out_hbm.at[pl.ds(base, rows_per_w)])

    return gk(table, idx_flat)


def _body_c(g_ref, qg_ref, pos_ref, wk_ref, wv_ref, out_ref):
    g = g_ref[0]
    kg = jnp.dot(g, wk_ref[...], preferred_element_type=jnp.float32)
    vg = jnp.dot(g, wv_ref[...], preferred_element_type=jnp.float32)
    qg = qg_ref[0]

    w = jax.lax.dot_general(qg, kg, (((1,), (1,)), ((), ())),
                            preferred_element_type=jnp.float32)
    w = w * (1.0 / math.sqrt(D))
    colmask = jax.lax.broadcasted_iota(jnp.int32, (S, S), 1) < NSEL
    w = jnp.where(colmask, w, -1e30)
    w = w - jnp.max(w, axis=1, keepdims=True)
    p = jnp.exp(w)
    a = p / jnp.sum(p, axis=1, keepdims=True)
    og = jnp.dot(a, vg, preferred_element_type=jnp.float32)

    iota_s = jax.lax.broadcasted_iota(jnp.int32, (S, TB), 0)

    def scatter(r, _):
        pr = pos_ref[0, pl.ds(r, 1), :]
        chunk = (pr == iota_s).astype(jnp.float32)
        out_ref[0, pl.ds(r * TB, TB), :] = jax.lax.dot_general(
            chunk, og, (((0,), (0,)), ((), ())),
            preferred_element_type=jnp.float32)
        return 0

    jax.lax.fori_loop(0, NT, scatter, 0)


def _stage_c(g, qg, pos, Wk, Wv, interpret):
    return pl.pallas_call(
        _body_c,
        grid=(B,),
        in_specs=[
            pl.BlockSpec((1, S, E), lambda b: (b, 0, 0)),
            pl.BlockSpec((1, S, D), lambda b: (b, 0, 0)),
            pl.BlockSpec((1, NT, TB), lambda b: (b, 0, 0)),
            pl.BlockSpec((E, D), lambda b: (0, 0)),
            pl.BlockSpec((E, D), lambda b: (0, 0)),
        ],
        out_specs=pl.BlockSpec((1, T, D), lambda b: (b, 0, 0)),
        out_shape=jax.ShapeDtypeStruct((B, T, D), jnp.float32),
        interpret=interpret,
    )(g, qg, pos, Wk, Wv)


def kernel(index, Wq, Wk, Wv):
    qg, pos, idxf = _stage_a(index, Wq, False)
    idx_flat = (idxf[:, 0, :].astype(jnp.int32)
                + (jnp.arange(B, dtype=jnp.int32) * T)[:, None]).reshape(B * S)
    g = _sc_gather(index.reshape(B * T, E), idx_flat)
    return _stage_c(g.reshape(B, S, E), qg, pos, Wk, Wv, False)


# speedup vs baseline: 1.7108x; 1.7108x over previous
import math

import jax
import jax.numpy as jnp
from jax.experimental import pallas as pl
from jax.experimental.pallas import tpu as pltpu

B = 2
T = 4096
E = 2048
D = 128
NT = 8
TB = T // NT
NSEL = int(0.1 * T)
S = 512
_F32_INF_BITS = 0x7F800000


def _body(idx_ref, w_ref, out_ref, qkv_s, nrm_s, pos_s, eq_s):
    t = pl.program_id(1)
    xb = idx_ref[0].astype(jnp.bfloat16)
    qkv = jnp.dot(xb, w_ref[...], preferred_element_type=jnp.float32)
    qkv_s[pl.ds(t * TB, TB), :] = qkv
    q = qkv[:, :D]
    ones_row = jnp.ones((1, D), jnp.float32)
    n2 = jax.lax.dot_general(ones_row, q * q, (((1,), (1,)), ((), ())),
                             preferred_element_type=jnp.float32, precision=jax.lax.Precision.HIGHEST)
    nrm_s[pl.ds(t, 1), :] = jnp.sqrt(n2)

    @pl.when(t == NT - 1)
    def _finish():
        norms = nrm_s[...]
        bits = jax.lax.bitcast_convert_type(norms, jnp.int32)

        def bs_body(_, carry):
            lo, hi = carry
            mid = lo + (hi - lo) // 2
            cnt = jnp.sum((bits > mid).astype(jnp.int32))
            big = cnt >= NSEL
            return jnp.where(big, mid, lo), jnp.where(big, hi, mid)

        lo, hi = jax.lax.fori_loop(
            0, 31, bs_body, (jnp.int32(-1), jnp.int32(_F32_INF_BITS)))
        thr = hi
        m_gt = bits > thr
        m_eq = bits == thr

        io_i = jax.lax.broadcasted_iota(jnp.int32, (TB, TB), 0)
        io_j = jax.lax.broadcasted_iota(jnp.int32, (TB, TB), 1)
        tri_l = (io_i <= io_j).astype(jnp.float32)
        ro_i = jax.lax.broadcasted_iota(jnp.int32, (NT, NT), 0)
        ro_j = jax.lax.broadcasted_iota(jnp.int32, (NT, NT), 1)
        tri_s = (ro_j < ro_i).astype(jnp.float32)

        def csum(mb):
            mf = mb.astype(jnp.float32)
            within = jnp.dot(mf, tri_l, preferred_element_type=jnp.float32, precision=jax.lax.Precision.HIGHEST)
            off = jnp.dot(tri_s, within[:, TB - 1:TB],
                          preferred_element_type=jnp.float32, precision=jax.lax.Precision.HIGHEST)
            return within + off

        n_gt = jnp.sum(m_gt.astype(jnp.int32))
        need = (NSEL - n_gt).astype(jnp.float32)
        sel = m_gt | (m_eq & (csum(m_eq) <= need))
        pos = csum(sel) - 1.0
        pos_s[...] = jnp.where(sel, pos.astype(jnp.int32), -1)

        iota_s = jax.lax.broadcasted_iota(jnp.int32, (S, TB), 0)

        def build_gather(r, acc):
            pr = pos_s[pl.ds(r, 1), :]
            chunk = (pr == iota_s).astype(jnp.float32)
            eq_s[pl.ds(r, 1)] = chunk[None]
            return acc + jnp.dot(chunk, qkv_s[pl.ds(r * TB, TB), :],
                                 preferred_element_type=jnp.float32)

        gg = jax.lax.fori_loop(0, NT, build_gather,
                               jnp.zeros((S, 3 * D), jnp.float32))
        qg, kg, vg = gg[:, :D], gg[:, D:2 * D], gg[:, 2 * D:]

        w = jax.lax.dot_general(qg, kg, (((1,), (1,)), ((), ())),
                                preferred_element_type=jnp.float32)
        w = w * (1.0 / math.sqrt(D))
        colmask = jax.lax.broadcasted_iota(jnp.int32, (S, S), 1) < NSEL
        w = jnp.where(colmask, w, -1e30)
        w = w - jnp.max(w, axis=1, keepdims=True)
        p = jnp.exp(w)
        a = p / jnp.sum(p, axis=1, keepdims=True)
        og = jnp.dot(a, vg, preferred_element_type=jnp.float32)

        def scatter(r, _):
            out_ref[0, pl.ds(r * TB, TB), :] = jax.lax.dot_general(
                eq_s[pl.ds(r, 1)][0], og, (((0,), (0,)), ((), ())),
                preferred_element_type=jnp.float32)
            return 0

        jax.lax.fori_loop(0, NT, scatter, 0)


def _run(index, W, interpret):
    return pl.pallas_call(
        _body,
        grid=(B, NT),
        in_specs=[
            pl.BlockSpec((1, TB, E), lambda b, t: (b, t, 0)),
            pl.BlockSpec((E, 3 * D), lambda b, t: (0, 0)),
        ],
        out_specs=pl.BlockSpec((1, T, D), lambda b, t: (b, 0, 0)),
        out_shape=jax.ShapeDtypeStruct((B, T, D), jnp.float32),
        scratch_shapes=[
            pltpu.VMEM((T, 3 * D), jnp.float32),
            pltpu.VMEM((NT, TB), jnp.float32),
            pltpu.VMEM((NT, TB), jnp.int32),
            pltpu.VMEM((NT, S, TB), jnp.float32),
        ],
        compiler_params=pltpu.CompilerParams(
            vmem_limit_bytes=112 * 1024 * 1024),
        interpret=interpret,
    )(index, W)


def kernel(index, Wq, Wk, Wv):
    W = jnp.concatenate([Wq, Wk, Wv], axis=1).astype(jnp.bfloat16)
    return _run(index, W, False)


# speedup vs baseline: 1.7146x; 1.0022x over previous
import math

import jax
import jax.numpy as jnp
from jax.experimental import pallas as pl
from jax.experimental.pallas import tpu as pltpu

B = 2
T = 4096
E = 2048
D = 128
NT = 8
TB = T // NT
NSEL = int(0.1 * T)
S = 512
_F32_INF_BITS = 0x7F800000


def _body(idx_ref, w_ref, out_ref, qkv_s, nrm_s, pos_s):
    t = pl.program_id(1)
    xb = idx_ref[0].astype(jnp.bfloat16)
    qkv = jnp.dot(xb, w_ref[...], preferred_element_type=jnp.float32)
    qkv_s[pl.ds(t * TB, TB), :] = qkv
    q = qkv[:, :D]
    ones_row = jnp.ones((1, D), jnp.float32)
    n2 = jax.lax.dot_general(ones_row, q * q, (((1,), (1,)), ((), ())),
                             preferred_element_type=jnp.float32, precision=jax.lax.Precision.HIGHEST)
    nrm_s[pl.ds(t, 1), :] = jnp.sqrt(n2)

    @pl.when(t == NT - 1)
    def _finish():
        norms = nrm_s[...]
        bits = jax.lax.bitcast_convert_type(norms, jnp.int32)

        def bs_body(_, carry):
            lo, hi = carry
            mid = lo + (hi - lo) // 2
            cnt = jnp.sum((bits > mid).astype(jnp.int32))
            big = cnt >= NSEL
            return jnp.where(big, mid, lo), jnp.where(big, hi, mid)

        lo, hi = jax.lax.fori_loop(
            0, 31, bs_body, (jnp.int32(-1), jnp.int32(_F32_INF_BITS)))
        thr = hi
        m_gt = bits > thr
        m_eq = bits == thr

        io_i = jax.lax.broadcasted_iota(jnp.int32, (TB, TB), 0)
        io_j = jax.lax.broadcasted_iota(jnp.int32, (TB, TB), 1)
        tri_l = (io_i <= io_j).astype(jnp.float32)
        ro_i = jax.lax.broadcasted_iota(jnp.int32, (NT, NT), 0)
        ro_j = jax.lax.broadcasted_iota(jnp.int32, (NT, NT), 1)
        tri_s = (ro_j < ro_i).astype(jnp.float32)

        def csum(mb):
            mf = mb.astype(jnp.float32)
            within = jnp.dot(mf, tri_l, preferred_element_type=jnp.float32, precision=jax.lax.Precision.HIGHEST)
            off = jnp.dot(tri_s, within[:, TB - 1:TB],
                          preferred_element_type=jnp.float32, precision=jax.lax.Precision.HIGHEST)
            return within + off

        n_gt = jnp.sum(m_gt.astype(jnp.int32))
        need = (NSEL - n_gt).astype(jnp.float32)
        sel = m_gt | (m_eq & (csum(m_eq) <= need))
        pos = csum(sel) - 1.0
        pos_s[...] = jnp.where(sel, pos.astype(jnp.int32), -1)

        iota_s = jax.lax.broadcasted_iota(jnp.int32, (S, TB), 0)

        def build_gather(r, acc):
            pr = pos_s[pl.ds(r, 1), :]
            chunk = (pr == iota_s).astype(jnp.float32)
            return acc + jnp.dot(chunk, qkv_s[pl.ds(r * TB, TB), :],
                                 preferred_element_type=jnp.float32)

        gg = jax.lax.fori_loop(0, NT, build_gather,
                               jnp.zeros((S, 3 * D), jnp.float32))
        qg, kg, vg = gg[:, :D], gg[:, D:2 * D], gg[:, 2 * D:]

        w = jax.lax.dot_general(qg, kg, (((1,), (1,)), ((), ())),
                                preferred_element_type=jnp.float32)
        w = w * (1.0 / math.sqrt(D))
        colmask = jax.lax.broadcasted_iota(jnp.int32, (S, S), 1) < NSEL
        w = jnp.where(colmask, w, -1e30)
        w = w - jnp.max(w, axis=1, keepdims=True)
        p = jnp.exp(w)
        a = p / jnp.sum(p, axis=1, keepdims=True)
        og = jnp.dot(a, vg, preferred_element_type=jnp.float32)

        def scatter(r, _):
            pr = pos_s[pl.ds(r, 1), :]
            chunk = (pr == iota_s).astype(jnp.float32)
            out_ref[0, pl.ds(r * TB, TB), :] = jax.lax.dot_general(
                chunk, og, (((0,), (0,)), ((), ())),
                preferred_element_type=jnp.float32)
            return 0

        jax.lax.fori_loop(0, NT, scatter, 0)


def _run(index, W, interpret):
    return pl.pallas_call(
        _body,
        grid=(B, NT),
        in_specs=[
            pl.BlockSpec((1, TB, E), lambda b, t: (b, t, 0)),
            pl.BlockSpec((E, 3 * D), lambda b, t: (0, 0)),
        ],
        out_specs=pl.BlockSpec((1, T, D), lambda b, t: (b, 0, 0)),
        out_shape=jax.ShapeDtypeStruct((B, T, D), jnp.float32),
        scratch_shapes=[
            pltpu.VMEM((T, 3 * D), jnp.float32),
            pltpu.VMEM((NT, TB), jnp.float32),
            pltpu.VMEM((NT, TB), jnp.int32),
        ],
        compiler_params=pltpu.CompilerParams(
            vmem_limit_bytes=112 * 1024 * 1024),
        interpret=interpret,
    )(index, W)


def kernel(index, Wq, Wk, Wv):
    W = jnp.concatenate([Wq, Wk, Wv], axis=1).astype(jnp.bfloat16)
    return _run(index, W, False)
